# Initial kernel scaffold; baseline (speedup 1.0000x reference)
import jax
import jax.numpy as jnp
from jax.experimental import pallas as pl


def kernel(inputs, drug_table, cmpd_table):
    raise NotImplementedError("write your pallas kernel here")


# speedup vs baseline: 1.1851x; 1.1851x over previous
import functools

import jax
import jax.numpy as jnp
from jax import lax
from jax.experimental import pallas as pl
from jax.experimental.pallas import tpu as pltpu
from jax.experimental.pallas import tpu_sc as plsc

_PERM_DNUMS = lax.GatherDimensionNumbers(
    offset_dims=(), collapsed_slice_dims=(0,), start_index_map=(0,))


def _permute(v, idx):
    return lax.gather(v, idx[:, None], _PERM_DNUMS, slice_sizes=(1,),
                      mode=lax.GatherScatterMode.PROMISE_IN_BOUNDS)


H = 128
LANES = 16
NC = 2
NS = 16
NW = NC * NS
CHUNK = 128


@functools.lru_cache(maxsize=None)
def _build(batch: int):
    assert batch % (NW * CHUNK) == 0
    kpw = batch // (NW * CHUNK)
    ppw = kpw * CHUNK
    mesh = plsc.VectorSubcoreMesh(core_axis_name="c", subcore_axis_name="s")

    @functools.partial(
        pl.kernel,
        mesh=mesh,
        out_type=jax.ShapeDtypeStruct((batch,), jnp.float32),
        scratch_types=[
            pltpu.VMEM((kpw, CHUNK), jnp.int32),
            pltpu.VMEM((kpw, CHUNK), jnp.int32),
            pltpu.VMEM((CHUNK, H), jnp.float32),
            pltpu.VMEM((CHUNK, H), jnp.float32),
            pltpu.VMEM((ppw,), jnp.float32),
            pltpu.SemaphoreType.DMA,
        ],
    )
    def sc_kernel(idx0_hbm, idx1_hbm, drug_hbm, cmpd_hbm, out_hbm,
                  idx0_v, idx1_v, drows_v, crows_v, out_v, sem):
        wid = lax.axis_index("s") * NC + lax.axis_index("c")
        row0 = wid * kpw
        pltpu.sync_copy(idx0_hbm.at[pl.ds(row0, kpw)], idx0_v)
        pltpu.sync_copy(idx1_hbm.at[pl.ds(row0, kpw)], idx1_v)

        for j in range(kpw):
            cp_d = pltpu.async_copy(drug_hbm.at[idx0_v.at[j]], drows_v, sem)
            cp_c = pltpu.async_copy(cmpd_hbm.at[idx1_v.at[j]], crows_v, sem)
            cp_d.wait()
            cp_c.wait()

            lane = lax.broadcasted_iota(jnp.int32, (LANES,), 0)

            for g in range(CHUNK // LANES):
                def pair_body(t, vec, g=g):
                    b = g * LANES + t
                    acc = (drows_v[b, pl.ds(0, LANES)]
                           * crows_v[b, pl.ds(0, LANES)])
                    for i in range(1, H // LANES):
                        acc = acc + (drows_v[b, pl.ds(i * LANES, LANES)]
                                     * crows_v[b, pl.ds(i * LANES, LANES)])
                    for sh in (8, 4, 2, 1):
                        acc = acc + _permute(acc, jnp.bitwise_xor(lane, sh))
                    return jnp.where(lane == t, acc, vec)

                vec = lax.fori_loop(0, LANES, pair_body,
                                    jnp.zeros((LANES,), jnp.float32))
                out_v[pl.ds(j * CHUNK + g * LANES, LANES)] = vec

        pltpu.sync_copy(out_v, out_hbm.at[pl.ds(wid * ppw, ppw)])

    return sc_kernel


def kernel(inputs, drug_table, cmpd_table):
    batch = inputs.shape[0]
    idx = inputs.astype(jnp.int32)
    idx0 = idx[:, 0].reshape(batch // CHUNK, CHUNK)
    idx1 = idx[:, 1].reshape(batch // CHUNK, CHUNK)
    out = _build(batch)(idx0, idx1, drug_table, cmpd_table)
    return out.reshape(batch, 1, 1)
